# baseline (speedup 1.0000x reference)
import functools

import jax
import jax.numpy as jnp
from jax import lax
from jax.experimental import pallas as pl
from jax.experimental.pallas import tpu as pltpu
from jax.experimental.pallas import tpu_sc as plsc

N_NODES = 50000
N_EDGES = 800000
IN_CH = 100
HID = 128
NUM_CLASSES = 2
NUM_GRAPHS = 64
EPS = 1e-5

R = 512
NP = 50176
NC = 2
NS = 16
NQ = 4
QR = NP // NQ
RA = 12560
EB = 64
SB = 1024
ET = 819200
ZR = 16


def _layer_norm(h, g, b, n_valid):
    if n_valid == HID:
        m = jnp.sum(h, axis=1, keepdims=True) / n_valid
        d = h - m
        v = jnp.sum(d * d, axis=1, keepdims=True) / n_valid
    else:
        mask = (lax.broadcasted_iota(jnp.int32, (1, HID), 1) < n_valid).astype(h.dtype)
        m = jnp.sum(h * mask, axis=1, keepdims=True) / n_valid
        d = (h - m) * mask
        v = jnp.sum(d * d, axis=1, keepdims=True) / n_valid
    return (h - m) * lax.rsqrt(v + EPS) * g + b


def _tc_pre_kernel(h_ref, g_ref, b_ref, w_ref, o_ref, *, n_valid):
    h = _layer_norm(h_ref[...], g_ref[...], b_ref[...], n_valid)
    o_ref[...] = jnp.dot(h, w_ref[...], preferred_element_type=jnp.float32)


def _tc_mid_kernel(a_ref, bp_ref, g_ref, b_ref, w_ref, o_ref):
    h = jnp.maximum(a_ref[...] + bp_ref[...], 0.0)
    h = _layer_norm(h, g_ref[...], b_ref[...], HID)
    o_ref[...] = jnp.dot(h, w_ref[...], preferred_element_type=jnp.float32)


def _tc_pool_kernel(a_ref, bp_ref, seg_ref, wf_ref, bf_ref, o_ref,
                    sums_ref, cnts_ref):
    i = pl.program_id(0)

    @pl.when(i == 0)
    def _():
        sums_ref[...] = jnp.zeros_like(sums_ref)
        cnts_ref[...] = jnp.zeros_like(cnts_ref)

    h = jnp.maximum(a_ref[...] + bp_ref[...], 0.0)
    seg = seg_ref[...]
    onehot = (seg == lax.broadcasted_iota(jnp.int32, (1, NUM_GRAPHS), 1)
              ).astype(jnp.float32)
    dn = (((0,), (0,)), ((), ()))
    sums_ref[...] += lax.dot_general(onehot, h, dn,
                                     preferred_element_type=jnp.float32)
    cnts_ref[...] += lax.dot_general(onehot, jnp.ones_like(h), dn,
                                     preferred_element_type=jnp.float32)

    @pl.when(i == pl.num_programs(0) - 1)
    def _():
        pooled = sums_ref[...] / jnp.maximum(cnts_ref[...], 1.0)
        o_ref[...] = jnp.dot(pooled, wf_ref[...],
                             preferred_element_type=jnp.float32) + bf_ref[...]


def _sc_agg_kernel(h, srcp, dstp, wp, out,
                   accum, src_sb0, dst_sb0, w_sb0, src_sb1, dst_sb1, w_sb1,
                   csrcf, cdstf, cwf,
                   src0, src1, dstl0, dstl1, wq0, wq1,
                   rows0, rows1, zbuf, sem0, sem1, ssem0, ssem1,
                   stsem0, stsem1):
    cid = lax.axis_index("c")
    sid = lax.axis_index("s")
    srcs = (src0, src1)
    dstls = (dstl0, dstl1)
    wqs = (wq0, wq1)
    rows = (rows0, rows1)
    sems = (sem0, sem1)
    ssems = (ssem0, ssem1)
    sbufs = ((src_sb0, dst_sb0, w_sb0), (src_sb1, dst_sb1, w_sb1))
    stsems = (stsem0, stsem1)
    iota16 = lax.iota(jnp.int32, 16)

    def zb_body(i, carry):
        for hh in range(HID // 16):
            zbuf[i, pl.ds(hh * 16, 16)] = jnp.zeros((16,), jnp.float32)
        return carry

    lax.fori_loop(0, ZR, zb_body, 0)

    for qq in range(NQ // NC):
        q = cid * (NQ // NC) + qq
        qlo = q * QR

        r0 = sid * (RA // NS)
        for k in range(RA // NS // ZR):
            pltpu.sync_copy(zbuf, accum.at[pl.ds(r0 + k * ZR, ZR)])
        rem = RA // NS % ZR
        if rem:
            pltpu.sync_copy(zbuf.at[pl.ds(0, rem)],
                            accum.at[pl.ds(r0 + (RA // NS // ZR) * ZR, rem)])
        plsc.subcore_barrier()

        ebase = sid * (ET // NS)

        def stage_batch(b, s):
            for g in range(EB // 16):
                idx = b * EB + (g * 16) + iota16
                osl = pl.ds(g * 16, 16)
                srcs[s][osl] = plsc.load_gather(csrcf, [idx])
                dstls[s][osl] = plsc.load_gather(cdstf, [idx])
                wqs[s][osl] = plsc.load_gather(cwf, [idx])
            return pltpu.async_copy(h.at[srcs[s]], rows[s], sems[s])

        def wait_mul_scatter(s):
            pltpu.make_async_copy(h.at[srcs[s]], rows[s], sems[s]).wait()

            def mul_body(e, carry2):
                we = plsc.load_gather(wqs[s], [jnp.full((16,), e, jnp.int32)])
                for hh in range(HID // 16):
                    sl = pl.ds(hh * 16, 16)
                    rows[s][e, sl] = rows[s][e, sl] * we
                return carry2

            lax.fori_loop(0, EB, mul_body, 0, unroll=2)
            pltpu.async_copy(rows[s], accum.at[dstls[s]], ssems[s], add=True)

        def wait_scatter(s):
            pltpu.make_async_copy(rows[s], accum.at[dstls[s]], ssems[s]).wait()

        def stage_sb(sbi, ss):
            e0 = ebase + sbi * SB
            sr, ds_, wr = sbufs[ss]
            d0 = pltpu.async_copy(srcp.at[pl.ds(e0, SB)], sr, stsems[ss])
            d1 = pltpu.async_copy(dstp.at[pl.ds(e0, SB)], ds_, stsems[ss])
            d2 = pltpu.async_copy(wp.at[pl.ds(e0, SB)], wr, stsems[ss])
            return (d0, d1, d2)

        def wait_sb(sbi, ss):
            e0 = ebase + sbi * SB
            sr, ds_, wr = sbufs[ss]
            pltpu.make_async_copy(srcp.at[pl.ds(e0, SB)], sr, stsems[ss]).wait()
            pltpu.make_async_copy(dstp.at[pl.ds(e0, SB)], ds_, stsems[ss]).wait()
            pltpu.make_async_copy(wp.at[pl.ds(e0, SB)], wr, stsems[ss]).wait()

        def process_sb(sbi, ss):
            src_sb, dst_sb, w_sb = sbufs[ss]
            wait_sb(sbi, ss)
            fill = jnp.int32(0)
            for j2 in range(SB // 16):
                sl = pl.ds(j2 * 16, 16)
                d16 = dst_sb[sl]
                m = (d16 >= qlo) & (d16 < qlo + QR)
                cum = plsc.cumsum(m.astype(jnp.int32))
                pos = fill + cum - 1
                plsc.store_scatter(csrcf, [pos], src_sb[sl], mask=m)
                plsc.store_scatter(cdstf, [pos], d16 - qlo, mask=m)
                plsc.store_scatter(cwf, [pos], w_sb[sl], mask=m)
                fill = fill + lax.reduce_max(cum, (0,))
            @pl.when(sbi + 2 < ET // NS // SB)
            def _():
                stage_sb(sbi + 2, ss)
            dummy16 = jnp.int32(QR) + iota16
            for g in range(EB // 16):
                pidx = fill + (g * 16) + iota16
                plsc.store_scatter(csrcf, [pidx], iota16 * 32)
                plsc.store_scatter(cdstf, [pidx], dummy16)
                plsc.store_scatter(cwf, [pidx], jnp.zeros((16,), jnp.float32))
            nb = (fill + (EB - 1)) // EB

            @pl.when(nb > 0)
            def _():
                stage_batch(0, 0)

            def pair_body(k, carry2):
                b0 = 2 * k
                b1 = 2 * k + 1

                @pl.when(b1 < nb)
                def _():
                    @pl.when(b1 >= 2)
                    def _():
                        wait_scatter(1)
                    stage_batch(b1, 1)

                @pl.when(b0 < nb)
                def _():
                    wait_mul_scatter(0)

                @pl.when(b1 < nb)
                def _():
                    wait_mul_scatter(1)

                @pl.when(b1 + 1 < nb)
                def _():
                    wait_scatter(0)
                    stage_batch(b1 + 1, 0)

                return carry2

            lax.fori_loop(0, (nb + 1) // 2, pair_body, 0)

            @pl.when(nb >= 1)
            def _():
                wait_scatter(0)

            @pl.when(nb >= 2)
            def _():
                wait_scatter(1)

        stage_sb(0, 0)
        stage_sb(1, 1)

        def sbpair_body(k, carry):
            process_sb(2 * k, 0)
            process_sb(2 * k + 1, 1)
            return carry

        lax.fori_loop(0, ET // NS // SB // 2, sbpair_body, 0)
        plsc.subcore_barrier()

        pltpu.sync_copy(accum.at[pl.ds(sid * (QR // NS), QR // NS)],
                        out.at[pl.ds(qlo + sid * (QR // NS), QR // NS)])
        plsc.subcore_barrier()


def _sc_aggregate(h, srcp, dstp, wp):
    mesh = plsc.VectorSubcoreMesh(core_axis_name="c", subcore_axis_name="s",
                                  num_cores=NC, num_subcores=NS)
    f = pl.kernel(
        _sc_agg_kernel,
        out_type=jax.ShapeDtypeStruct((NP, HID), jnp.float32),
        mesh=mesh,
        compiler_params=pltpu.CompilerParams(needs_layout_passes=False),
        scratch_types=[
            pltpu.MemorySpace.VMEM_SHARED((RA, HID), jnp.float32),
            pltpu.MemorySpace.VMEM((SB,), jnp.int32),
            pltpu.MemorySpace.VMEM((SB,), jnp.int32),
            pltpu.MemorySpace.VMEM((SB,), jnp.float32),
            pltpu.MemorySpace.VMEM((SB,), jnp.int32),
            pltpu.MemorySpace.VMEM((SB,), jnp.int32),
            pltpu.MemorySpace.VMEM((SB,), jnp.float32),
            pltpu.MemorySpace.VMEM((SB + 2 * EB,), jnp.int32),
            pltpu.MemorySpace.VMEM((SB + 2 * EB,), jnp.int32),
            pltpu.MemorySpace.VMEM((SB + 2 * EB,), jnp.float32),
            pltpu.MemorySpace.VMEM((EB,), jnp.int32),
            pltpu.MemorySpace.VMEM((EB,), jnp.int32),
            pltpu.MemorySpace.VMEM((EB,), jnp.int32),
            pltpu.MemorySpace.VMEM((EB,), jnp.int32),
            pltpu.MemorySpace.VMEM((EB,), jnp.float32),
            pltpu.MemorySpace.VMEM((EB,), jnp.float32),
            pltpu.MemorySpace.VMEM((EB, HID), jnp.float32),
            pltpu.MemorySpace.VMEM((EB, HID), jnp.float32),
            pltpu.MemorySpace.VMEM((ZR, HID), jnp.float32),
            pltpu.SemaphoreType.DMA,
            pltpu.SemaphoreType.DMA,
            pltpu.SemaphoreType.DMA,
            pltpu.SemaphoreType.DMA,
            pltpu.SemaphoreType.DMA,
            pltpu.SemaphoreType.DMA,
        ],
    )
    return f(h, srcp, dstp, wp)


def _tc_pre(h, g, b, w, n_valid):
    grid = (NP // R,)
    return pl.pallas_call(
        functools.partial(_tc_pre_kernel, n_valid=n_valid),
        grid=grid,
        in_specs=[
            pl.BlockSpec((R, HID), lambda i: (i, 0)),
            pl.BlockSpec((1, HID), lambda i: (0, 0)),
            pl.BlockSpec((1, HID), lambda i: (0, 0)),
            pl.BlockSpec((HID, HID), lambda i: (0, 0)),
        ],
        out_specs=pl.BlockSpec((R, HID), lambda i: (i, 0)),
        out_shape=jax.ShapeDtypeStruct((NP, HID), jnp.float32),
    )(h, g, b, w)


def _tc_mid(a, bp, g, b, w):
    grid = (NP // R,)
    return pl.pallas_call(
        _tc_mid_kernel,
        grid=grid,
        in_specs=[
            pl.BlockSpec((R, HID), lambda i: (i, 0)),
            pl.BlockSpec((1, HID), lambda i: (0, 0)),
            pl.BlockSpec((1, HID), lambda i: (0, 0)),
            pl.BlockSpec((1, HID), lambda i: (0, 0)),
            pl.BlockSpec((HID, HID), lambda i: (0, 0)),
        ],
        out_specs=pl.BlockSpec((R, HID), lambda i: (i, 0)),
        out_shape=jax.ShapeDtypeStruct((NP, HID), jnp.float32),
    )(a, bp, g, b, w)


def _tc_pool(a, bp, seg, wf, bf):
    grid = (NP // R,)
    return pl.pallas_call(
        _tc_pool_kernel,
        grid=grid,
        in_specs=[
            pl.BlockSpec((R, HID), lambda i: (i, 0)),
            pl.BlockSpec((1, HID), lambda i: (0, 0)),
            pl.BlockSpec((R, 1), lambda i: (i, 0)),
            pl.BlockSpec((HID, HID), lambda i: (0, 0)),
            pl.BlockSpec((1, HID), lambda i: (0, 0)),
        ],
        out_specs=pl.BlockSpec((NUM_GRAPHS, HID), lambda i: (0, 0)),
        out_shape=jax.ShapeDtypeStruct((NUM_GRAPHS, HID), jnp.float32),
        scratch_shapes=[
            pltpu.VMEM((NUM_GRAPHS, HID), jnp.float32),
            pltpu.VMEM((NUM_GRAPHS, HID), jnp.float32),
        ],
    )(a, bp, seg, wf, bf)


def kernel(x, edge_index, edge_weight, batch,
           ln0_g, ln0_b, W0, b0,
           ln1_g, ln1_b, W1, b1,
           ln2_g, ln2_b, W2, b2,
           Wf, bf):
    f32 = jnp.float32

    x_pad = jnp.zeros((NP, HID), f32).at[:N_NODES, :IN_CH].set(x)
    seg = jnp.full((NP, 1), NUM_GRAPHS, jnp.int32)
    seg = seg.at[:N_NODES, 0].set(batch.astype(jnp.int32))

    src = edge_index[0].astype(jnp.int32)
    dst = edge_index[1].astype(jnp.int32)
    npad = ET - N_EDGES
    fill = (jnp.arange(npad, dtype=jnp.int32) * 61) % N_NODES
    srcp = jnp.concatenate([src, fill])
    dstp = jnp.concatenate([dst, fill])
    wp = jnp.concatenate([edge_weight.astype(f32), jnp.zeros((npad,), f32)])

    def pad_rows(w):
        return jnp.zeros((HID, HID), f32).at[:w.shape[0], :w.shape[1]].set(w)

    W0p = pad_rows(W0)
    g0 = jnp.zeros((1, HID), f32).at[0, :IN_CH].set(ln0_g)
    b0v = jnp.zeros((1, HID), f32).at[0, :IN_CH].set(ln0_b)
    row = lambda v: v.reshape(1, HID)
    wfp = jnp.zeros((HID, HID), f32).at[:, :NUM_CLASSES].set(Wf)
    bfp = jnp.zeros((1, HID), f32).at[0, :NUM_CLASSES].set(bf)

    h = _tc_pre(x_pad, g0, b0v, W0p, IN_CH)
    a = _sc_aggregate(h, srcp, dstp, wp)
    h = _tc_mid(a, row(b0), row(ln1_g), row(ln1_b), W1)
    a = _sc_aggregate(h, srcp, dstp, wp)
    h = _tc_mid(a, row(b1), row(ln2_g), row(ln2_b), W2)
    a = _sc_aggregate(h, srcp, dstp, wp)
    out = _tc_pool(a, row(b2), seg, wfp, bfp)
    return out[:, :NUM_CLASSES]


# speedup vs baseline: 1.0140x; 1.0140x over previous
import functools

import jax
import jax.numpy as jnp
from jax import lax
from jax.experimental import pallas as pl
from jax.experimental.pallas import tpu as pltpu
from jax.experimental.pallas import tpu_sc as plsc

N_NODES = 50000
N_EDGES = 800000
IN_CH = 100
HID = 128
NUM_CLASSES = 2
NUM_GRAPHS = 64
EPS = 1e-5

R = 512
NP = 50176
NC = 2
NS = 16
NQ = 4
QR = NP // NQ
RA = 12560
EB = 64
SB = 1024
ET = 819200
ZR = 16


def _layer_norm(h, g, b, n_valid):
    if n_valid == HID:
        m = jnp.sum(h, axis=1, keepdims=True) / n_valid
        d = h - m
        v = jnp.sum(d * d, axis=1, keepdims=True) / n_valid
    else:
        mask = (lax.broadcasted_iota(jnp.int32, (1, HID), 1) < n_valid).astype(h.dtype)
        m = jnp.sum(h * mask, axis=1, keepdims=True) / n_valid
        d = (h - m) * mask
        v = jnp.sum(d * d, axis=1, keepdims=True) / n_valid
    return (h - m) * lax.rsqrt(v + EPS) * g + b


def _tc_pre_kernel(h_ref, g_ref, b_ref, w_ref, o_ref, *, n_valid):
    h = _layer_norm(h_ref[...], g_ref[...], b_ref[...], n_valid)
    o_ref[...] = jnp.dot(h, w_ref[...], preferred_element_type=jnp.float32)


def _tc_mid_kernel(a_ref, bp_ref, g_ref, b_ref, w_ref, o_ref):
    h = jnp.maximum(a_ref[...] + bp_ref[...], 0.0)
    h = _layer_norm(h, g_ref[...], b_ref[...], HID)
    o_ref[...] = jnp.dot(h, w_ref[...], preferred_element_type=jnp.float32)


def _tc_pool_kernel(a_ref, bp_ref, seg_ref, wf_ref, bf_ref, o_ref,
                    sums_ref, cnts_ref):
    i = pl.program_id(0)

    @pl.when(i == 0)
    def _():
        sums_ref[...] = jnp.zeros_like(sums_ref)
        cnts_ref[...] = jnp.zeros_like(cnts_ref)

    h = jnp.maximum(a_ref[...] + bp_ref[...], 0.0)
    seg = seg_ref[...]
    onehot = (seg == lax.broadcasted_iota(jnp.int32, (1, NUM_GRAPHS), 1)
              ).astype(jnp.float32)
    dn = (((0,), (0,)), ((), ()))
    sums_ref[...] += lax.dot_general(onehot, h, dn,
                                     preferred_element_type=jnp.float32)
    cnts_ref[...] += lax.dot_general(onehot, jnp.ones_like(h), dn,
                                     preferred_element_type=jnp.float32)

    @pl.when(i == pl.num_programs(0) - 1)
    def _():
        pooled = sums_ref[...] / jnp.maximum(cnts_ref[...], 1.0)
        o_ref[...] = jnp.dot(pooled, wf_ref[...],
                             preferred_element_type=jnp.float32) + bf_ref[...]


def _sc_agg_kernel(h, srcp, dstp, wp, out,
                   accum, src_sb0, dst_sb0, w_sb0, src_sb1, dst_sb1, w_sb1,
                   csrcf, cdstf, cwf,
                   src0, src1, dstl0, dstl1, wq0, wq1,
                   rows0, rows1, zbuf, sem0, sem1, ssem0, ssem1,
                   stsem0, stsem1):
    cid = lax.axis_index("c")
    sid = lax.axis_index("s")
    srcs = (src0, src1)
    dstls = (dstl0, dstl1)
    wqs = (wq0, wq1)
    rows = (rows0, rows1)
    sems = (sem0, sem1)
    ssems = (ssem0, ssem1)
    sbufs = ((src_sb0, dst_sb0, w_sb0), (src_sb1, dst_sb1, w_sb1))
    stsems = (stsem0, stsem1)
    iota16 = lax.iota(jnp.int32, 16)

    def zb_body(i, carry):
        for hh in range(HID // 16):
            zbuf[i, pl.ds(hh * 16, 16)] = jnp.zeros((16,), jnp.float32)
        return carry

    lax.fori_loop(0, ZR, zb_body, 0)

    for qq in range(NQ // NC):
        q = cid * (NQ // NC) + qq
        qlo = q * QR

        r0 = sid * (RA // NS)
        for k in range(RA // NS // ZR):
            pltpu.sync_copy(zbuf, accum.at[pl.ds(r0 + k * ZR, ZR)])
        rem = RA // NS % ZR
        if rem:
            pltpu.sync_copy(zbuf.at[pl.ds(0, rem)],
                            accum.at[pl.ds(r0 + (RA // NS // ZR) * ZR, rem)])
        plsc.subcore_barrier()

        ebase = sid * (ET // NS)

        def stage_batch(b, s):
            for g in range(EB // 16):
                idx = b * EB + (g * 16) + iota16
                osl = pl.ds(g * 16, 16)
                srcs[s][osl] = plsc.load_gather(csrcf, [idx])
                dstls[s][osl] = plsc.load_gather(cdstf, [idx])
                wqs[s][osl] = plsc.load_gather(cwf, [idx])
            return pltpu.async_copy(h.at[srcs[s]], rows[s], sems[s])

        def wait_mul_scatter(s):
            pltpu.make_async_copy(h.at[srcs[s]], rows[s], sems[s]).wait()

            def mul_body(e, carry2):
                we = plsc.load_gather(wqs[s], [jnp.full((16,), e, jnp.int32)])
                for hh in range(HID // 16):
                    sl = pl.ds(hh * 16, 16)
                    rows[s][e, sl] = rows[s][e, sl] * we
                return carry2

            lax.fori_loop(0, EB, mul_body, 0, unroll=2)
            pltpu.async_copy(rows[s], accum.at[dstls[s]], ssems[s], add=True)

        def wait_scatter(s):
            pltpu.make_async_copy(rows[s], accum.at[dstls[s]], ssems[s]).wait()

        def stage_sb(sbi, ss):
            e0 = ebase + sbi * SB
            sr, ds_, wr = sbufs[ss]
            d0 = pltpu.async_copy(srcp.at[pl.ds(e0, SB)], sr, stsems[ss])
            d1 = pltpu.async_copy(dstp.at[pl.ds(e0, SB)], ds_, stsems[ss])
            d2 = pltpu.async_copy(wp.at[pl.ds(e0, SB)], wr, stsems[ss])
            return (d0, d1, d2)

        def wait_sb(sbi, ss):
            e0 = ebase + sbi * SB
            sr, ds_, wr = sbufs[ss]
            pltpu.make_async_copy(srcp.at[pl.ds(e0, SB)], sr, stsems[ss]).wait()
            pltpu.make_async_copy(dstp.at[pl.ds(e0, SB)], ds_, stsems[ss]).wait()
            pltpu.make_async_copy(wp.at[pl.ds(e0, SB)], wr, stsems[ss]).wait()

        def process_sb(sbi, ss):
            src_sb, dst_sb, w_sb = sbufs[ss]
            wait_sb(sbi, ss)
            fill = jnp.int32(0)
            for j2 in range(SB // 16):
                sl = pl.ds(j2 * 16, 16)
                d16 = dst_sb[sl]
                m = (d16 >= qlo) & (d16 < qlo + QR)
                cum = plsc.cumsum(m.astype(jnp.int32))
                pos = fill + cum - 1
                plsc.store_scatter(csrcf, [pos], src_sb[sl], mask=m)
                plsc.store_scatter(cdstf, [pos], d16 - qlo, mask=m)
                plsc.store_scatter(cwf, [pos], w_sb[sl], mask=m)
                fill = fill + lax.reduce_max(cum, (0,))
            @pl.when(sbi + 2 < ET // NS // SB)
            def _():
                stage_sb(sbi + 2, ss)
            dummy16 = jnp.int32(QR) + iota16
            for g in range(EB // 16):
                pidx = fill + (g * 16) + iota16
                plsc.store_scatter(csrcf, [pidx], iota16 * 32)
                plsc.store_scatter(cdstf, [pidx], dummy16)
                plsc.store_scatter(cwf, [pidx], jnp.zeros((16,), jnp.float32))
            nb = (fill + (EB - 1)) // EB

            @pl.when(nb > 0)
            def _():
                stage_batch(0, 0)

            def pair_body(k, carry2):
                b0 = 2 * k
                b1 = 2 * k + 1

                @pl.when(b1 < nb)
                def _():
                    @pl.when(b1 >= 2)
                    def _():
                        wait_scatter(1)
                    stage_batch(b1, 1)

                @pl.when(b0 < nb)
                def _():
                    wait_mul_scatter(0)

                @pl.when(b1 + 1 < nb)
                def _():
                    wait_scatter(0)
                    stage_batch(b1 + 1, 0)

                @pl.when(b1 < nb)
                def _():
                    wait_mul_scatter(1)

                return carry2

            lax.fori_loop(0, (nb + 1) // 2, pair_body, 0)

            @pl.when(nb >= 1)
            def _():
                wait_scatter(0)

            @pl.when(nb >= 2)
            def _():
                wait_scatter(1)

        stage_sb(0, 0)
        stage_sb(1, 1)

        def sbpair_body(k, carry):
            process_sb(2 * k, 0)
            process_sb(2 * k + 1, 1)
            return carry

        lax.fori_loop(0, ET // NS // SB // 2, sbpair_body, 0)
        plsc.subcore_barrier()

        pltpu.sync_copy(accum.at[pl.ds(sid * (QR // NS), QR // NS)],
                        out.at[pl.ds(qlo + sid * (QR // NS), QR // NS)])
        plsc.subcore_barrier()


def _sc_aggregate(h, srcp, dstp, wp):
    mesh = plsc.VectorSubcoreMesh(core_axis_name="c", subcore_axis_name="s",
                                  num_cores=NC, num_subcores=NS)
    f = pl.kernel(
        _sc_agg_kernel,
        out_type=jax.ShapeDtypeStruct((NP, HID), jnp.float32),
        mesh=mesh,
        compiler_params=pltpu.CompilerParams(needs_layout_passes=False),
        scratch_types=[
            pltpu.MemorySpace.VMEM_SHARED((RA, HID), jnp.float32),
            pltpu.MemorySpace.VMEM((SB,), jnp.int32),
            pltpu.MemorySpace.VMEM((SB,), jnp.int32),
            pltpu.MemorySpace.VMEM((SB,), jnp.float32),
            pltpu.MemorySpace.VMEM((SB,), jnp.int32),
            pltpu.MemorySpace.VMEM((SB,), jnp.int32),
            pltpu.MemorySpace.VMEM((SB,), jnp.float32),
            pltpu.MemorySpace.VMEM((SB + 2 * EB,), jnp.int32),
            pltpu.MemorySpace.VMEM((SB + 2 * EB,), jnp.int32),
            pltpu.MemorySpace.VMEM((SB + 2 * EB,), jnp.float32),
            pltpu.MemorySpace.VMEM((EB,), jnp.int32),
            pltpu.MemorySpace.VMEM((EB,), jnp.int32),
            pltpu.MemorySpace.VMEM((EB,), jnp.int32),
            pltpu.MemorySpace.VMEM((EB,), jnp.int32),
            pltpu.MemorySpace.VMEM((EB,), jnp.float32),
            pltpu.MemorySpace.VMEM((EB,), jnp.float32),
            pltpu.MemorySpace.VMEM((EB, HID), jnp.float32),
            pltpu.MemorySpace.VMEM((EB, HID), jnp.float32),
            pltpu.MemorySpace.VMEM((ZR, HID), jnp.float32),
            pltpu.SemaphoreType.DMA,
            pltpu.SemaphoreType.DMA,
            pltpu.SemaphoreType.DMA,
            pltpu.SemaphoreType.DMA,
            pltpu.SemaphoreType.DMA,
            pltpu.SemaphoreType.DMA,
        ],
    )
    return f(h, srcp, dstp, wp)


def _tc_pre(h, g, b, w, n_valid):
    grid = (NP // R,)
    return pl.pallas_call(
        functools.partial(_tc_pre_kernel, n_valid=n_valid),
        grid=grid,
        in_specs=[
            pl.BlockSpec((R, HID), lambda i: (i, 0)),
            pl.BlockSpec((1, HID), lambda i: (0, 0)),
            pl.BlockSpec((1, HID), lambda i: (0, 0)),
            pl.BlockSpec((HID, HID), lambda i: (0, 0)),
        ],
        out_specs=pl.BlockSpec((R, HID), lambda i: (i, 0)),
        out_shape=jax.ShapeDtypeStruct((NP, HID), jnp.float32),
    )(h, g, b, w)


def _tc_mid(a, bp, g, b, w):
    grid = (NP // R,)
    return pl.pallas_call(
        _tc_mid_kernel,
        grid=grid,
        in_specs=[
            pl.BlockSpec((R, HID), lambda i: (i, 0)),
            pl.BlockSpec((1, HID), lambda i: (0, 0)),
            pl.BlockSpec((1, HID), lambda i: (0, 0)),
            pl.BlockSpec((1, HID), lambda i: (0, 0)),
            pl.BlockSpec((HID, HID), lambda i: (0, 0)),
        ],
        out_specs=pl.BlockSpec((R, HID), lambda i: (i, 0)),
        out_shape=jax.ShapeDtypeStruct((NP, HID), jnp.float32),
    )(a, bp, g, b, w)


def _tc_pool(a, bp, seg, wf, bf):
    grid = (NP // R,)
    return pl.pallas_call(
        _tc_pool_kernel,
        grid=grid,
        in_specs=[
            pl.BlockSpec((R, HID), lambda i: (i, 0)),
            pl.BlockSpec((1, HID), lambda i: (0, 0)),
            pl.BlockSpec((R, 1), lambda i: (i, 0)),
            pl.BlockSpec((HID, HID), lambda i: (0, 0)),
            pl.BlockSpec((1, HID), lambda i: (0, 0)),
        ],
        out_specs=pl.BlockSpec((NUM_GRAPHS, HID), lambda i: (0, 0)),
        out_shape=jax.ShapeDtypeStruct((NUM_GRAPHS, HID), jnp.float32),
        scratch_shapes=[
            pltpu.VMEM((NUM_GRAPHS, HID), jnp.float32),
            pltpu.VMEM((NUM_GRAPHS, HID), jnp.float32),
        ],
    )(a, bp, seg, wf, bf)


def kernel(x, edge_index, edge_weight, batch,
           ln0_g, ln0_b, W0, b0,
           ln1_g, ln1_b, W1, b1,
           ln2_g, ln2_b, W2, b2,
           Wf, bf):
    f32 = jnp.float32

    x_pad = jnp.zeros((NP, HID), f32).at[:N_NODES, :IN_CH].set(x)
    seg = jnp.full((NP, 1), NUM_GRAPHS, jnp.int32)
    seg = seg.at[:N_NODES, 0].set(batch.astype(jnp.int32))

    src = edge_index[0].astype(jnp.int32)
    dst = edge_index[1].astype(jnp.int32)
    npad = ET - N_EDGES
    fill = (jnp.arange(npad, dtype=jnp.int32) * 61) % N_NODES
    srcp = jnp.concatenate([src, fill])
    dstp = jnp.concatenate([dst, fill])
    wp = jnp.concatenate([edge_weight.astype(f32), jnp.zeros((npad,), f32)])

    def pad_rows(w):
        return jnp.zeros((HID, HID), f32).at[:w.shape[0], :w.shape[1]].set(w)

    W0p = pad_rows(W0)
    g0 = jnp.zeros((1, HID), f32).at[0, :IN_CH].set(ln0_g)
    b0v = jnp.zeros((1, HID), f32).at[0, :IN_CH].set(ln0_b)
    row = lambda v: v.reshape(1, HID)
    wfp = jnp.zeros((HID, HID), f32).at[:, :NUM_CLASSES].set(Wf)
    bfp = jnp.zeros((1, HID), f32).at[0, :NUM_CLASSES].set(bf)

    h = _tc_pre(x_pad, g0, b0v, W0p, IN_CH)
    a = _sc_aggregate(h, srcp, dstp, wp)
    h = _tc_mid(a, row(b0), row(ln1_g), row(ln1_b), W1)
    a = _sc_aggregate(h, srcp, dstp, wp)
    h = _tc_mid(a, row(b1), row(ln2_g), row(ln2_b), W2)
    a = _sc_aggregate(h, srcp, dstp, wp)
    out = _tc_pool(a, row(b2), seg, wfp, bfp)
    return out[:, :NUM_CLASSES]
